# baseline (speedup 1.0000x reference)
import functools

import jax
import jax.numpy as jnp
from jax import lax
from jax.experimental import pallas as pl
from jax.experimental.pallas import tpu as pltpu
from jax.experimental.pallas import tpu_sc as plsc

B, L, K = 16, 512, 64
START_IDX, END_IDX, PAD_IDX = 61, 62, 63

NC, NS, LANES = 2, 16, 16
NW = NC * NS
HALF = L // 2


def _scan_body(scores_t_ref, wsl_col_ref, transition_ref, out_u_ref):
    trans = transition_ref[:, :]
    max_t = jnp.max(trans)
    exp_ts = jnp.exp(trans - max_t)
    wsl_col = wsl_col_ref[:, :]

    a0_log = trans[START_IDX:START_IDX + 1, :] + scores_t_ref[0]
    m0 = jnp.max(a0_log, axis=1, keepdims=True)
    a = jnp.exp(a0_log - m0)
    off = m0
    m_prev = jnp.max(a, axis=1, keepdims=True)

    def one_step(t, state):
        a, m_prev, off, last_a, last_off = state
        s = jnp.dot(a, exp_ts, preferred_element_type=jnp.float32)
        g = jnp.exp(scores_t_ref[t]) * (1.0 / m_prev)
        a_new = s * g
        off_new = off + (jnp.log(m_prev) + max_t)
        is_last = (wsl_col - 1) == t
        last_a = jnp.where(is_last, a_new, last_a)
        last_off = jnp.where(is_last, off_new, last_off)
        m_new = jnp.max(a_new, axis=1, keepdims=True)
        return a_new, m_new, off_new, last_a, last_off

    state = (a, m_prev, off, a, off)
    for t in range(1, 16):
        state = one_step(t, state)

    def body16(i, state):
        for j in range(16):
            state = one_step(16 * i + j, state)
        return state

    _, _, _, last_a, last_off = jax.lax.fori_loop(1, L // 16, body16, state)

    v = jnp.dot(last_a, exp_ts, preferred_element_type=jnp.float32)
    oh_end = (jax.lax.broadcasted_iota(jnp.int32, (B, K), 1) == END_IDX)
    picked = jnp.sum(jnp.where(oh_end, v, 0.0), axis=1, keepdims=True)
    ub = jnp.log(picked) + last_off + max_t
    out_u_ref[:, :] = jnp.sum(ub, axis=0, keepdims=True)


def _labeled_sc(lstm_flat, tags, prev, wsl, trans_flat,
                out_hbm, lstm_v, tags_v, prev_v, wsl_v, trans_v, acc_v):
    wid = lax.axis_index("s") * NC + lax.axis_index("c")
    b = wid // 2
    h = wid % 2

    pltpu.sync_copy(lstm_flat.at[b, pl.ds(h * HALF * K, HALF * K)], lstm_v)
    pltpu.sync_copy(tags.at[b, pl.ds(h * HALF, HALF)], tags_v)
    pltpu.sync_copy(prev.at[b, pl.ds(h * HALF, HALF)], prev_v)
    pltpu.sync_copy(wsl, wsl_v)
    pltpu.sync_copy(trans_flat, trans_v)

    bvec = jnp.full((LANES,), b, jnp.int32)
    len_vec = plsc.load_gather(wsl_v, [bvec])
    iota16 = lax.iota(jnp.int32, LANES)

    def round_g(g, acc):
        t_loc = iota16 + g * LANES
        tag = tags_v[pl.ds(g * LANES, LANES)]
        prv = prev_v[pl.ds(g * LANES, LANES)]
        tv = plsc.load_gather(trans_v, [prv * K + tag])
        ev = plsc.load_gather(lstm_v, [t_loc * K + tag])
        t_glob = t_loc + h * HALF
        maskf = jnp.where(t_glob < len_vec, 1.0, 0.0)
        return acc + (tv + ev) * maskf

    acc = lax.fori_loop(0, HALF // LANES, round_g,
                        jnp.zeros((LANES,), jnp.float32))

    lend = len_vec - 1
    ll = jnp.minimum(jnp.maximum(lend - h * HALF, 0), HALF - 1)
    last_tag = plsc.load_gather(tags_v, [ll])
    end_v = plsc.load_gather(trans_v, [last_tag * K + END_IDX])
    in_half = (lend >= h * HALF) & (lend < (h + 1) * HALF) & (iota16 == 0)
    acc = acc + jnp.where(in_half, end_v, 0.0)

    acc_v[...] = acc
    pltpu.sync_copy(acc_v, out_hbm.at[wid])


@jax.jit
def kernel(lstm_scores, word_seq_lens, tags, mask, transition):
    scores_t = jnp.transpose(lstm_scores, (1, 0, 2))
    wsl_col = word_seq_lens.reshape(B, 1)

    prev = jnp.concatenate(
        [jnp.full((B, 1), START_IDX, dtype=tags.dtype), tags[:, :-1]], axis=1)
    lstm_flat = lstm_scores.reshape(B, L * K)
    trans_flat = transition.reshape(K * K)

    sc_kernel = functools.partial(
        pl.kernel,
        out_type=jax.ShapeDtypeStruct((NW, LANES), jnp.float32),
        mesh=plsc.VectorSubcoreMesh(core_axis_name="c", subcore_axis_name="s"),
        compiler_params=pltpu.CompilerParams(needs_layout_passes=False),
        scratch_types=[
            pltpu.VMEM((HALF * K,), jnp.float32),
            pltpu.VMEM((HALF,), jnp.int32),
            pltpu.VMEM((HALF,), jnp.int32),
            pltpu.VMEM((B,), jnp.int32),
            pltpu.VMEM((K * K,), jnp.float32),
            pltpu.VMEM((LANES,), jnp.float32),
        ],
    )(_labeled_sc)
    partials = sc_kernel(lstm_flat, tags, prev, word_seq_lens, trans_flat)

    out_u = pl.pallas_call(
        _scan_body,
        out_shape=jax.ShapeDtypeStruct((1, 1), jnp.float32),
    )(scores_t, wsl_col, transition)

    labeled = jnp.sum(partials)
    return (out_u.reshape(()), labeled)


# speedup vs baseline: 1.1974x; 1.1974x over previous
import functools

import jax
import jax.numpy as jnp
from jax.experimental import pallas as pl

B, L, K = 16, 512, 64
START_IDX, END_IDX, PAD_IDX = 61, 62, 63


def _crf_body(scores_t_ref, tags_t_ref, prev_t_ref, maskf_t_ref,
              wsl_col_ref, wsl_row_ref, transition_ref,
              out_u_ref, out_l_ref):
    trans = transition_ref[:, :]
    max_t = jnp.max(trans)
    exp_ts = jnp.exp(trans - max_t)

    wsl_col = wsl_col_ref[:, :]
    wsl_row = wsl_row_ref[:, :]

    def labeled_chunk(t0):
        tags_c = tags_t_ref[pl.ds(t0, 4), :]
        prev_c = prev_t_ref[pl.ds(t0, 4), :]
        maskf_c = maskf_t_ref[pl.ds(t0, 4), :]
        scores_c = scores_t_ref[pl.ds(t0, 4), :, :]
        iota_k3 = jax.lax.broadcasted_iota(jnp.int32, (4, B, K), 2)
        oh_tag = (tags_c[:, :, None] == iota_k3).astype(jnp.float32)
        oh_prev = (prev_c[:, :, None] == iota_k3).astype(jnp.float32)
        u = jnp.dot(jnp.reshape(oh_prev, (4 * B, K)), trans,
                    preferred_element_type=jnp.float32,
                    precision=jax.lax.Precision.HIGHEST)
        u3 = jnp.reshape(u, (4, B, K))
        contrib = jnp.sum(oh_tag * (u3 + scores_c), axis=2) * maskf_c
        iota_t = jax.lax.broadcasted_iota(jnp.int32, (4, B), 0) + t0
        is_last = (iota_t == (wsl_row - 1)).astype(jnp.int32)
        lt_part = tags_c * is_last
        return contrib, lt_part

    a0_log = trans[START_IDX:START_IDX + 1, :] + scores_t_ref[0]
    m0 = jnp.max(a0_log, axis=1, keepdims=True)
    a = jnp.exp(a0_log - m0)
    off = m0
    m_prev = jnp.max(a, axis=1, keepdims=True)

    def one_step(t, state):
        a, m_prev, off, last_a, last_off = state
        s = jnp.dot(a, exp_ts, preferred_element_type=jnp.float32)
        g = jnp.exp(scores_t_ref[t]) * (1.0 / m_prev)
        a_new = s * g
        off_new = off + (jnp.log(m_prev) + max_t)
        is_last = (wsl_col - 1) == t
        last_a = jnp.where(is_last, a_new, last_a)
        last_off = jnp.where(is_last, off_new, last_off)
        m_new = jnp.max(a_new, axis=1, keepdims=True)
        return a_new, m_new, off_new, last_a, last_off

    state = (a, m_prev, off, a, off)
    state = one_step(1, state)
    state = one_step(2, state)
    state = one_step(3, state)

    for t in range(4, 32):
        state = one_step(t, state)

    acc0 = None
    lt0 = None
    for t0 in range(0, 32, 4):
        c, ltp = labeled_chunk(t0)
        acc0 = c if acc0 is None else acc0 + c
        lt0 = ltp if lt0 is None else lt0 + ltp

    def body32(i, carry):
        state, acc, lt = carry
        for j in range(32):
            state = one_step(32 * i + j, state)
        for j0 in range(0, 32, 4):
            c, ltp = labeled_chunk(32 * i + j0)
            acc = acc + c
            lt = lt + ltp
        return state, acc, lt

    (_, _, _, last_a, last_off), acc, lt = jax.lax.fori_loop(
        1, L // 32, body32, (state, acc0, lt0))

    v = jnp.dot(last_a, exp_ts, preferred_element_type=jnp.float32)
    oh_end = (jax.lax.broadcasted_iota(jnp.int32, (B, K), 1) == END_IDX)
    picked = jnp.sum(jnp.where(oh_end, v, 0.0), axis=1, keepdims=True)
    ub = jnp.log(picked) + last_off + max_t
    out_u_ref[:, :] = jnp.sum(ub, axis=0, keepdims=True)

    seq_sum = jnp.sum(acc)
    last_tag = jnp.sum(lt, axis=0, keepdims=True)
    iota_kb = jax.lax.broadcasted_iota(jnp.int32, (K, B), 0)
    oh_last = (last_tag == iota_kb).astype(jnp.float32)
    cnt = jnp.sum(oh_last, axis=1, keepdims=True)
    end_sum = jnp.sum(cnt * trans[:, END_IDX:END_IDX + 1])

    total = seq_sum + end_sum
    out_l_ref[:, :] = jnp.reshape(total, (1, 1))


@jax.jit
def kernel(lstm_scores, word_seq_lens, tags, mask, transition):
    scores_t = jnp.transpose(lstm_scores, (1, 0, 2))
    tags_t = jnp.transpose(tags, (1, 0))
    prev = jnp.concatenate(
        [jnp.full((B, 1), START_IDX, dtype=tags.dtype), tags[:, :-1]], axis=1)
    prev_t = jnp.transpose(prev, (1, 0))
    maskf_t = jnp.transpose(mask.astype(jnp.float32), (1, 0))
    wsl_col = word_seq_lens.reshape(B, 1)
    wsl_row = word_seq_lens.reshape(1, B)

    out_u, out_l = pl.pallas_call(
        _crf_body,
        out_shape=[
            jax.ShapeDtypeStruct((1, 1), jnp.float32),
            jax.ShapeDtypeStruct((1, 1), jnp.float32),
        ],
    )(scores_t, tags_t, prev_t, maskf_t, wsl_col, wsl_row, transition)
    return (out_u.reshape(()), out_l.reshape(()))


# speedup vs baseline: 1.2036x; 1.0052x over previous
import functools

import jax
import jax.numpy as jnp
from jax.experimental import pallas as pl

B, L, K = 16, 512, 64
START_IDX, END_IDX, PAD_IDX = 61, 62, 63


def _crf_body(scores_t_ref, tags_t_ref, prev_t_ref, maskf_t_ref,
              wsl_col_ref, wsl_row_ref, transition_ref,
              out_u_ref, out_l_ref):
    trans = transition_ref[:, :]
    max_t = jnp.max(trans)
    exp_ts = jnp.exp(trans - max_t)

    wsl_col = wsl_col_ref[:, :]
    wsl_row = wsl_row_ref[:, :]

    def labeled_chunk(t0):
        tags_c = tags_t_ref[pl.ds(t0, 4), :]
        prev_c = prev_t_ref[pl.ds(t0, 4), :]
        maskf_c = maskf_t_ref[pl.ds(t0, 4), :]
        scores_c = scores_t_ref[pl.ds(t0, 4), :, :]
        iota_k3 = jax.lax.broadcasted_iota(jnp.int32, (4, B, K), 2)
        oh_tag = (tags_c[:, :, None] == iota_k3).astype(jnp.float32)
        oh_prev = (prev_c[:, :, None] == iota_k3).astype(jnp.float32)
        u = jnp.dot(jnp.reshape(oh_prev, (4 * B, K)), trans,
                    preferred_element_type=jnp.float32,
                    precision=jax.lax.Precision.HIGHEST)
        u3 = jnp.reshape(u, (4, B, K))
        contrib = jnp.sum(oh_tag * (u3 + scores_c), axis=2) * maskf_c
        iota_t = jax.lax.broadcasted_iota(jnp.int32, (4, B), 0) + t0
        is_last = (iota_t == (wsl_row - 1)).astype(jnp.int32)
        lt_part = tags_c * is_last
        return contrib, lt_part

    a0_log = trans[START_IDX:START_IDX + 1, :] + scores_t_ref[0]
    m0 = jnp.max(a0_log, axis=1, keepdims=True)
    a = jnp.exp(a0_log - m0)
    off = m0
    m_prev = jnp.max(a, axis=1, keepdims=True)

    def one_step(t, state):
        a, m_prev, off, last_a, last_off = state
        s = jnp.dot(a, exp_ts, preferred_element_type=jnp.float32)
        g = jnp.exp(scores_t_ref[t]) * (1.0 / m_prev)
        a_new = s * g
        off_new = off + (jnp.log(m_prev) + max_t)
        is_last = (wsl_col - 1) == t
        last_a = jnp.where(is_last, a_new, last_a)
        last_off = jnp.where(is_last, off_new, last_off)
        m_new = jnp.max(a_new, axis=1, keepdims=True)
        return a_new, m_new, off_new, last_a, last_off

    state = (a, m_prev, off, a, off)
    state = one_step(1, state)
    state = one_step(2, state)
    state = one_step(3, state)

    for t in range(4, 64):
        state = one_step(t, state)

    acc0 = None
    lt0 = None
    for t0 in range(0, 64, 4):
        c, ltp = labeled_chunk(t0)
        acc0 = c if acc0 is None else acc0 + c
        lt0 = ltp if lt0 is None else lt0 + ltp

    def body64(i, carry):
        state, acc, lt = carry
        for j in range(64):
            state = one_step(64 * i + j, state)
        for j0 in range(0, 64, 4):
            c, ltp = labeled_chunk(64 * i + j0)
            acc = acc + c
            lt = lt + ltp
        return state, acc, lt

    (_, _, _, last_a, last_off), acc, lt = jax.lax.fori_loop(
        1, L // 64, body64, (state, acc0, lt0))

    v = jnp.dot(last_a, exp_ts, preferred_element_type=jnp.float32)
    oh_end = (jax.lax.broadcasted_iota(jnp.int32, (B, K), 1) == END_IDX)
    picked = jnp.sum(jnp.where(oh_end, v, 0.0), axis=1, keepdims=True)
    ub = jnp.log(picked) + last_off + max_t
    out_u_ref[:, :] = jnp.sum(ub, axis=0, keepdims=True)

    seq_sum = jnp.sum(acc)
    last_tag = jnp.sum(lt, axis=0, keepdims=True)
    iota_kb = jax.lax.broadcasted_iota(jnp.int32, (K, B), 0)
    oh_last = (last_tag == iota_kb).astype(jnp.float32)
    cnt = jnp.sum(oh_last, axis=1, keepdims=True)
    end_sum = jnp.sum(cnt * trans[:, END_IDX:END_IDX + 1])

    total = seq_sum + end_sum
    out_l_ref[:, :] = jnp.reshape(total, (1, 1))


@jax.jit
def kernel(lstm_scores, word_seq_lens, tags, mask, transition):
    scores_t = jnp.transpose(lstm_scores, (1, 0, 2))
    tags_t = jnp.transpose(tags, (1, 0))
    prev = jnp.concatenate(
        [jnp.full((B, 1), START_IDX, dtype=tags.dtype), tags[:, :-1]], axis=1)
    prev_t = jnp.transpose(prev, (1, 0))
    maskf_t = jnp.transpose(mask.astype(jnp.float32), (1, 0))
    wsl_col = word_seq_lens.reshape(B, 1)
    wsl_row = word_seq_lens.reshape(1, B)

    out_u, out_l = pl.pallas_call(
        _crf_body,
        out_shape=[
            jax.ShapeDtypeStruct((1, 1), jnp.float32),
            jax.ShapeDtypeStruct((1, 1), jnp.float32),
        ],
    )(scores_t, tags_t, prev_t, maskf_t, wsl_col, wsl_row, transition)
    return (out_u.reshape(()), out_l.reshape(()))
